# Initial kernel scaffold; baseline (speedup 1.0000x reference)
import jax
import jax.numpy as jnp
from jax.experimental import pallas as pl


def kernel(esm_features, geometric_features, node_coords, edge_index, edge_attr, batch_index, params):
    raise NotImplementedError("write your pallas kernel here")


# speedup vs baseline: 12.5076x; 12.5076x over previous
import functools

import jax
import jax.numpy as jnp
from jax import lax
from jax.experimental import pallas as pl
from jax.experimental.pallas import tpu as pltpu
from jax.experimental.pallas import tpu_sc as plsc

N_NODES = 10000
N_EDGES = 160000
N_GRAPHS = 32
ESM_DIM = 640
HID = 128
OUT_DIM = 256
N_LAYERS = 3

TROW = 144
EFD = 132
N_PAD = 10240
CH = 128
NCHUNK = N_EDGES // CH
NW = 32
KMAX = (NCHUNK + NW - 1) // NW
RPT = N_PAD // 16
ZROWS = 64

_f32 = jnp.float32


def _erf(x):
    a1, a2, a3, a4, a5 = (0.254829592, -0.284496736, 1.421413741,
                          -1.453152027, 1.061405429)
    p = 0.3275911
    ax = jnp.abs(x)
    t = 1.0 / (1.0 + p * ax)
    poly = ((((a5 * t + a4) * t + a3) * t + a2) * t + a1) * t
    y = 1.0 - poly * jnp.exp(-ax * ax)
    return jnp.sign(x) * y


def _gelu(x):
    return 0.5 * x * (1.0 + _erf(x * 0.7071067811865476))


def _sigmoid(x):
    return 1.0 / (1.0 + jnp.exp(-x))


def _ln_gelu(s, lnw, lnb):
    mu = jnp.mean(s, axis=1, keepdims=True)
    var = jnp.mean((s - mu) ** 2, axis=1, keepdims=True)
    sn = (s - mu) * lax.rsqrt(var + 1e-5) * lnw + lnb
    return _gelu(sn)


def _geo_body(ca, cb, nxt, prv, out):
    n = ca.shape[1]
    lane = lax.broadcasted_iota(jnp.int32, (1, n), 1)

    def norm3(v):
        nn = jnp.sqrt(jnp.sum(v * v, axis=0, keepdims=True))
        return v / jnp.maximum(nn, 1e-12)

    def cross(a, b):
        ax, ay, az = a[0:1], a[1:2], a[2:3]
        bx, by, bz = b[0:1], b[1:2], b[2:3]
        return jnp.concatenate(
            [ay * bz - az * by, az * bx - ax * bz, ax * by - ay * bx], axis=0)

    cav, cbv, nxv, prv_v = ca[...], cb[...], nxt[...], prv[...]
    ca_cb = norm3(cbv - cav)
    ca_dir = jnp.where(lane == n - 1, 0.0, nxv - cav)
    ca_dir_n = norm3(ca_dir)
    normal_n = norm3(cross(ca_cb, ca_dir_n))
    tang = jnp.where((lane == 0) | (lane == n - 1), 0.0, nxv - prv_v)
    tang_n = norm3(tang)
    out[...] = jnp.concatenate([ca_cb, ca_dir_n, normal_n, tang_n], axis=0)


def _node_vectors(coords, cb):
    caT = coords.T
    cbT = cb.T
    nxtT = jnp.roll(coords, -1, axis=0).T
    prvT = jnp.roll(coords, 1, axis=0).T
    out = pl.pallas_call(
        _geo_body,
        out_shape=jax.ShapeDtypeStruct((12, N_NODES), _f32),
    )(caT, cbT, nxtT, prvT)
    return out.T


def _in_node_body(esm, geo9, w1, w2, b, out):
    out[...] = (jnp.dot(esm[...], w1[...], preferred_element_type=_f32)
                + jnp.dot(geo9[...], w2[...], preferred_element_type=_f32)
                + b[...])


def _node_proj(esm, geo9, w1, w2, b):
    blk = 2000
    grid = N_NODES // blk
    return pl.pallas_call(
        _in_node_body,
        grid=(grid,),
        in_specs=[
            pl.BlockSpec((blk, ESM_DIM), lambda i: (i, 0)),
            pl.BlockSpec((blk, 9), lambda i: (i, 0)),
            pl.BlockSpec((ESM_DIM, HID), lambda i: (0, 0)),
            pl.BlockSpec((9, HID), lambda i: (0, 0)),
            pl.BlockSpec((1, HID), lambda i: (0, 0)),
        ],
        out_specs=pl.BlockSpec((blk, HID), lambda i: (i, 0)),
        out_shape=jax.ShapeDtypeStruct((N_NODES, HID), _f32),
    )(esm, geo9, w1, w2, b)


def _edge_feat_body(e13, ev, w, b, out):
    es = jnp.dot(e13[...], w[...], preferred_element_type=_f32) + b[...]
    evv = ev[...]
    evn = jnp.sqrt(jnp.sum(evv * evv, axis=1, keepdims=True))
    out[...] = jnp.concatenate([es, evn, evv], axis=1)


def _edge_feat(e13, ev, w, b):
    blk = 2000
    grid = N_EDGES // blk
    return pl.pallas_call(
        _edge_feat_body,
        grid=(grid,),
        in_specs=[
            pl.BlockSpec((blk, 13), lambda i: (i, 0)),
            pl.BlockSpec((blk, 3), lambda i: (i, 0)),
            pl.BlockSpec((13, HID), lambda i: (0, 0)),
            pl.BlockSpec((1, HID), lambda i: (0, 0)),
        ],
        out_specs=pl.BlockSpec((blk, EFD), lambda i: (i, 0)),
        out_shape=jax.ShapeDtypeStruct((N_EDGES, EFD), _f32),
    )(e13, ev, w, b)


def _tables_body(ns, nv, s4, wcat, ts, td):
    nvv = nv[...]
    vn = jnp.sqrt(jnp.dot(nvv * nvv, s4[...], preferred_element_type=_f32))
    x = jnp.concatenate([ns[...], vn, nvv], axis=1)
    y = jnp.dot(x, wcat[...], preferred_element_type=_f32)
    ts[...] = y[:, :TROW]
    td[...] = y[:, TROW:]


def _make_tables(ns, nv, s4, wcat):
    blk = 2000
    grid = N_NODES // blk
    return pl.pallas_call(
        _tables_body,
        grid=(grid,),
        in_specs=[
            pl.BlockSpec((blk, HID), lambda i: (i, 0)),
            pl.BlockSpec((blk, 12), lambda i: (i, 0)),
            pl.BlockSpec((12, 4), lambda i: (0, 0)),
            pl.BlockSpec((TROW, 2 * TROW), lambda i: (0, 0)),
        ],
        out_specs=[
            pl.BlockSpec((blk, TROW), lambda i: (i, 0)),
            pl.BlockSpec((blk, TROW), lambda i: (i, 0)),
        ],
        out_shape=[
            jax.ShapeDtypeStruct((N_NODES, TROW), _f32),
            jax.ShapeDtypeStruct((N_NODES, TROW), _f32),
        ],
    )(ns, nv, s4, wcat)


def _edge_msg_body(ga, gb, ef, wedge, lnw, lnb, bexp, out):
    t = (ga[...] + gb[...]
         + jnp.dot(ef[...], wedge[...], preferred_element_type=_f32))
    s = t[:, :HID]
    g = t[:, HID:HID + 4]
    v = t[:, HID + 4:TROW]
    act = _ln_gelu(s, lnw[...], lnb[...])
    gate = jnp.dot(_sigmoid(g), bexp[...], preferred_element_type=_f32)
    z = jnp.zeros((t.shape[0], 4), _f32)
    out[...] = jnp.concatenate([act, v * gate, z], axis=1)


def _edge_msgs(ga, gb, ef, wedge, lnw, lnb, bexp):
    blk = 2000
    grid = N_EDGES // blk
    return pl.pallas_call(
        _edge_msg_body,
        grid=(grid,),
        in_specs=[
            pl.BlockSpec((blk, TROW), lambda i: (i, 0)),
            pl.BlockSpec((blk, TROW), lambda i: (i, 0)),
            pl.BlockSpec((blk, EFD), lambda i: (i, 0)),
            pl.BlockSpec((EFD, TROW), lambda i: (0, 0)),
            pl.BlockSpec((1, HID), lambda i: (0, 0)),
            pl.BlockSpec((1, HID), lambda i: (0, 0)),
            pl.BlockSpec((4, 12), lambda i: (0, 0)),
        ],
        out_specs=pl.BlockSpec((blk, TROW), lambda i: (i, 0)),
        out_shape=jax.ShapeDtypeStruct((N_EDGES, TROW), _f32),
    )(ga, gb, ef, wedge, lnw, lnb, bexp)


def _update_body(ns, nv, ag0, ag1, s8, wup, wexp, lnw, lnb, bexp,
                 ns_out, nv_out):
    nsv = ns[...]
    nvv = nv[...]
    ag = ag0[...] + ag1[...]
    uv = jnp.concatenate([nvv, ag[:, HID:HID + 12]], axis=1)
    vn = jnp.sqrt(jnp.dot(uv * uv, s8[...], preferred_element_type=_f32))
    x = jnp.concatenate([nsv, ag[:, :HID], vn], axis=1)
    y = jnp.dot(x, wup[...], preferred_element_type=_f32)
    act = _ln_gelu(y[:, :HID], lnw[...], lnb[...])
    gate = jnp.dot(_sigmoid(y[:, HID:HID + 4]), bexp[...],
                   preferred_element_type=_f32)
    vout = jnp.dot(uv, wexp[...], preferred_element_type=_f32) * gate
    ns_out[...] = nsv + act
    nv_out[...] = nvv + vout


def _update(ns, nv, ag0, ag1, s8, wup, wexp, lnw, lnb, bexp):
    blk = 2000
    grid = N_NODES // blk
    return pl.pallas_call(
        _update_body,
        grid=(grid,),
        in_specs=[
            pl.BlockSpec((blk, HID), lambda i: (i, 0)),
            pl.BlockSpec((blk, 12), lambda i: (i, 0)),
            pl.BlockSpec((blk, TROW), lambda i: (i, 0)),
            pl.BlockSpec((blk, TROW), lambda i: (i, 0)),
            pl.BlockSpec((24, 8), lambda i: (0, 0)),
            pl.BlockSpec((264, EFD), lambda i: (0, 0)),
            pl.BlockSpec((24, 12), lambda i: (0, 0)),
            pl.BlockSpec((1, HID), lambda i: (0, 0)),
            pl.BlockSpec((1, HID), lambda i: (0, 0)),
            pl.BlockSpec((4, 12), lambda i: (0, 0)),
        ],
        out_specs=[
            pl.BlockSpec((blk, HID), lambda i: (i, 0)),
            pl.BlockSpec((blk, 12), lambda i: (i, 0)),
        ],
        out_shape=[
            jax.ShapeDtypeStruct((N_NODES, HID), _f32),
            jax.ShapeDtypeStruct((N_NODES, 12), _f32),
        ],
    )(ns, nv, ag0, ag1, s8, wup, wexp, lnw, lnb, bexp)


def _out_body(ns, bi, wno, bno, wgo, bgo, upd, gemb, acc, cnt):
    i = pl.program_id(0)
    nsv = ns[...]
    upd[...] = jnp.dot(nsv, wno[...], preferred_element_type=_f32) + bno[...]
    gf = jnp.dot(nsv, wgo[...], preferred_element_type=_f32) + bgo[...]
    b = bi[0]
    gid = lax.broadcasted_iota(jnp.int32, (N_GRAPHS, 1), 0)
    oh = (b == gid).astype(_f32)

    @pl.when(i == 0)
    def _():
        acc[...] = jnp.zeros_like(acc)
        cnt[...] = jnp.zeros_like(cnt)

    acc[...] += jnp.dot(oh, gf, preferred_element_type=_f32)
    cnt[...] += jnp.broadcast_to(
        jnp.sum(oh, axis=1, keepdims=True), cnt.shape)

    @pl.when(i == pl.num_programs(0) - 1)
    def _():
        gemb[...] = acc[...] / jnp.maximum(cnt[:, :1], 1.0)


def _heads(ns, bi3, wno, bno, wgo, bgo):
    blk = 2000
    grid = N_NODES // blk
    return pl.pallas_call(
        _out_body,
        grid=(grid,),
        in_specs=[
            pl.BlockSpec((blk, HID), lambda i: (i, 0)),
            pl.BlockSpec((1, 1, blk), lambda i: (i, 0, 0)),
            pl.BlockSpec((HID, ESM_DIM), lambda i: (0, 0)),
            pl.BlockSpec((1, ESM_DIM), lambda i: (0, 0)),
            pl.BlockSpec((HID, OUT_DIM), lambda i: (0, 0)),
            pl.BlockSpec((1, OUT_DIM), lambda i: (0, 0)),
        ],
        out_specs=[
            pl.BlockSpec((blk, ESM_DIM), lambda i: (i, 0)),
            pl.BlockSpec((N_GRAPHS, OUT_DIM), lambda i: (0, 0)),
        ],
        out_shape=[
            jax.ShapeDtypeStruct((N_NODES, ESM_DIM), _f32),
            jax.ShapeDtypeStruct((N_GRAPHS, OUT_DIM), _f32),
        ],
        scratch_shapes=[
            pltpu.VMEM((N_GRAPHS, OUT_DIM), _f32),
            pltpu.VMEM((N_GRAPHS, HID), _f32),
        ],
    )(ns, bi3, wno, bno, wgo, bgo)


@functools.cache
def _sc_kernels():
    mesh = plsc.VectorSubcoreMesh(core_axis_name="c", subcore_axis_name="s",
                                  num_cores=2, num_subcores=16)

    @functools.partial(
        pl.kernel,
        out_type=(jax.ShapeDtypeStruct((N_EDGES, TROW), _f32),
                  jax.ShapeDtypeStruct((N_EDGES, TROW), _f32)),
        mesh=mesh,
        compiler_params=pltpu.CompilerParams(use_tc_tiling_on_sc=False),
        scratch_types=[
            pltpu.VMEM((CH,), jnp.int32),
            pltpu.VMEM((CH,), jnp.int32),
            pltpu.VMEM((CH, TROW), _f32),
            pltpu.VMEM((CH, TROW), _f32),
            pltpu.SemaphoreType.DMA,
            pltpu.SemaphoreType.DMA,
        ],
    )
    def _sc_gather(ts_hbm, td_hbm, src_hbm, dst_hbm, oa_hbm, ob_hbm,
                   ia, ib, ra, rb, sa, sb):
        wid = lax.axis_index("s") * 2 + lax.axis_index("c")

        def body(k, carry):
            cid = k * NW + wid

            @pl.when(cid < NCHUNK)
            def _():
                base = cid * CH
                pltpu.sync_copy(src_hbm.at[pl.ds(base, CH)], ia)
                ca = pltpu.async_copy(ts_hbm.at[ia], ra, sa)
                pltpu.sync_copy(dst_hbm.at[pl.ds(base, CH)], ib)
                cb = pltpu.async_copy(td_hbm.at[ib], rb, sb)
                ca.wait()
                cb.wait()
                pltpu.sync_copy(ra, oa_hbm.at[pl.ds(base, CH)])
                pltpu.sync_copy(rb, ob_hbm.at[pl.ds(base, CH)])

            return carry

        lax.fori_loop(0, KMAX, body, 0)

    @functools.partial(
        pl.kernel,
        out_type=jax.ShapeDtypeStruct((2 * N_PAD, TROW), _f32),
        mesh=mesh,
        compiler_params=pltpu.CompilerParams(use_tc_tiling_on_sc=False),
        scratch_types=[
            pltpu.VMEM((CH,), jnp.int32),
            pltpu.VMEM((CH, TROW), _f32),
            pltpu.VMEM((ZROWS, TROW), _f32),
            pltpu.VMEM_SHARED((N_PAD, TROW), _f32),
        ],
    )
    def _sc_scatter(m_hbm, dst_hbm, out_hbm, idxv, rows, zbuf, shared):
        c = lax.axis_index("c")
        s = lax.axis_index("s")
        wid = s * 2 + c

        def zero_body(i, carry):
            for j in range(TROW // 16):
                zbuf[i, pl.ds(j * 16, 16)] = jnp.zeros((16,), _f32)
            return carry

        lax.fori_loop(0, ZROWS, zero_body, 0)

        def zcopy(i, carry):
            pltpu.sync_copy(zbuf, shared.at[pl.ds(s * RPT + i * ZROWS, ZROWS)])
            return carry

        lax.fori_loop(0, RPT // ZROWS, zcopy, 0)
        plsc.subcore_barrier()

        def body(k, carry):
            cid = k * NW + wid

            @pl.when(cid < NCHUNK)
            def _():
                base = cid * CH
                pltpu.sync_copy(dst_hbm.at[pl.ds(base, CH)], idxv)
                pltpu.sync_copy(m_hbm.at[pl.ds(base, CH)], rows)
                pltpu.sync_copy(rows, shared.at[idxv], add=True)

            return carry

        lax.fori_loop(0, KMAX, body, 0)
        plsc.subcore_barrier()
        pltpu.sync_copy(shared.at[pl.ds(s * RPT, RPT)],
                        out_hbm.at[pl.ds(c * N_PAD + s * RPT, RPT)])

    return _sc_gather, _sc_scatter


def _sc_gather_pair(ts, td, src, dst):
    return _sc_kernels()[0](ts, td, src, dst)


def _sc_scatter_add(msgs, dst):
    out = _sc_kernels()[1](msgs, dst)
    ag = out.reshape(2, N_PAD, TROW)
    return ag[0, :N_NODES], ag[1, :N_NODES]


def _msg_tables_weight(m):
    wss_t = m['Wss'].T
    wvs_t = m['Wvs'].T
    wsv_t = m['Wsv'].T
    wvv = m['Wvv']
    i3 = jnp.eye(3, dtype=_f32)

    def half(lo, vlo):
        r_s = jnp.concatenate(
            [wss_t[lo:lo + HID], wsv_t[lo:lo + HID],
             jnp.zeros((HID, 12), _f32)], axis=1)
        r_vn = jnp.concatenate(
            [wvs_t[vlo:vlo + 4], jnp.zeros((4, 16), _f32)], axis=1)
        r_v = jnp.concatenate(
            [jnp.zeros((12, EFD), _f32),
             jnp.kron(wvv[:, vlo:vlo + 4].T, i3)], axis=1)
        return jnp.concatenate([r_s, r_vn, r_v], axis=0)

    return jnp.concatenate([half(0, 0), half(HID, 4)], axis=1)


def _msg_edge_weight(m):
    i3 = jnp.eye(3, dtype=_f32)
    r_s = jnp.concatenate(
        [m['Wss'][:, 2 * HID:].T, m['Wsv'][:, 2 * HID:].T,
         jnp.zeros((HID, 12), _f32)], axis=1)
    r_n = jnp.concatenate(
        [m['Wvs'][:, 8:9].T, jnp.zeros((1, 16), _f32)], axis=1)
    r_v = jnp.concatenate(
        [jnp.zeros((3, EFD), _f32),
         jnp.kron(m['Wvv'][:, 8:9].T, i3)], axis=1)
    return jnp.concatenate([r_s, r_n, r_v], axis=0)


def _upd_weights(u):
    i3 = jnp.eye(3, dtype=_f32)
    r_s = jnp.concatenate([u['Wss'].T, u['Wsv'].T], axis=1)
    r_n = jnp.concatenate([u['Wvs'].T, jnp.zeros((8, 4), _f32)], axis=1)
    wup = jnp.concatenate([r_s, r_n], axis=0)
    wexp = jnp.kron(u['Wvv'].T, i3)
    return wup, wexp


def kernel(esm_features, geometric_features, node_coords, edge_index,
           edge_attr, batch_index, params):
    f32 = _f32
    i3 = jnp.eye(3, dtype=f32)
    s4 = jnp.kron(jnp.eye(4, dtype=f32), jnp.ones((3, 1), f32))
    s8 = jnp.kron(jnp.eye(8, dtype=f32), jnp.ones((3, 1), f32))
    bexp = jnp.kron(jnp.eye(4, dtype=f32), jnp.ones((1, 3), f32))

    cb = geometric_features[:, 3:6]
    geo9 = geometric_features[:, 6:15]
    node_v = _node_vectors(node_coords, cb)

    node_s = _node_proj(
        esm_features, geo9,
        params['node_proj_W'][:, :ESM_DIM].T,
        params['node_proj_W'][:, ESM_DIM:].T,
        params['node_proj_b'][None, :])

    e13 = jnp.concatenate([edge_attr[:, 0:2], edge_attr[:, 5:]], axis=1)
    ev = edge_attr[:, 2:5]
    ef = _edge_feat(e13, ev, params['edge_proj_W'].T,
                    params['edge_proj_b'][None, :])

    src = edge_index[0]
    dst = edge_index[1]

    for lp in params['layers']:
        m = lp['msg']
        u = lp['upd']
        wcat = _msg_tables_weight(m)
        wedge = _msg_edge_weight(m)
        wup, wexp = _upd_weights(u)

        ts, td = _make_tables(node_s, node_v, s4, wcat)
        ga, gb = _sc_gather_pair(ts, td, src, dst)
        msgs = _edge_msgs(ga, gb, ef, wedge,
                          m['ln_w'][None, :], m['ln_b'][None, :], bexp)
        ag0, ag1 = _sc_scatter_add(msgs, dst)
        node_s, node_v = _update(node_s, node_v, ag0, ag1, s8, wup, wexp,
                                 u['ln_w'][None, :], u['ln_b'][None, :], bexp)

    bi3 = batch_index.reshape(5, 1, 2000)
    upd, gemb = _heads(node_s, bi3,
                       params['node_out_W'].T, params['node_out_b'][None, :],
                       params['graph_out_W'].T, params['graph_out_b'][None, :])
    return gemb, upd


# speedup vs baseline: 15.5819x; 1.2458x over previous
import functools

import jax
import jax.numpy as jnp
from jax import lax
from jax.experimental import pallas as pl
from jax.experimental.pallas import tpu as pltpu
from jax.experimental.pallas import tpu_sc as plsc

N_NODES = 10000
N_EDGES = 160000
N_GRAPHS = 32
ESM_DIM = 640
HID = 128
OUT_DIM = 256
N_LAYERS = 3

TROW = 144
GROW = 256
EFD = 132
N_PAD = 10240
CH = 128
NCHUNK = N_EDGES // CH
NW = 32
KMAX = (NCHUNK + NW - 1) // NW
RPT = N_PAD // 16
ZROWS = 64

_f32 = jnp.float32


def _erf(x):
    a1, a2, a3, a4, a5 = (0.254829592, -0.284496736, 1.421413741,
                          -1.453152027, 1.061405429)
    p = 0.3275911
    ax = jnp.abs(x)
    t = 1.0 / (1.0 + p * ax)
    poly = ((((a5 * t + a4) * t + a3) * t + a2) * t + a1) * t
    y = 1.0 - poly * jnp.exp(-ax * ax)
    return jnp.sign(x) * y


def _gelu(x):
    return 0.5 * x * (1.0 + _erf(x * 0.7071067811865476))


def _sigmoid(x):
    return 1.0 / (1.0 + jnp.exp(-x))


def _ln_gelu(s, lnw, lnb):
    mu = jnp.mean(s, axis=1, keepdims=True)
    var = jnp.mean((s - mu) ** 2, axis=1, keepdims=True)
    sn = (s - mu) * lax.rsqrt(var + 1e-5) * lnw + lnb
    return _gelu(sn)


def _geo_body(ca, cb, nxt, prv, out):
    n = ca.shape[1]
    lane = lax.broadcasted_iota(jnp.int32, (1, n), 1)

    def norm3(v):
        nn = jnp.sqrt(jnp.sum(v * v, axis=0, keepdims=True))
        return v / jnp.maximum(nn, 1e-12)

    def cross(a, b):
        ax, ay, az = a[0:1], a[1:2], a[2:3]
        bx, by, bz = b[0:1], b[1:2], b[2:3]
        return jnp.concatenate(
            [ay * bz - az * by, az * bx - ax * bz, ax * by - ay * bx], axis=0)

    cav, cbv, nxv, prv_v = ca[...], cb[...], nxt[...], prv[...]
    ca_cb = norm3(cbv - cav)
    ca_dir = jnp.where(lane == n - 1, 0.0, nxv - cav)
    ca_dir_n = norm3(ca_dir)
    normal_n = norm3(cross(ca_cb, ca_dir_n))
    tang = jnp.where((lane == 0) | (lane == n - 1), 0.0, nxv - prv_v)
    tang_n = norm3(tang)
    out[...] = jnp.concatenate([ca_cb, ca_dir_n, normal_n, tang_n], axis=0)


def _node_vectors(coords, cb):
    caT = coords.T
    cbT = cb.T
    nxtT = jnp.roll(coords, -1, axis=0).T
    prvT = jnp.roll(coords, 1, axis=0).T
    out = pl.pallas_call(
        _geo_body,
        out_shape=jax.ShapeDtypeStruct((12, N_NODES), _f32),
    )(caT, cbT, nxtT, prvT)
    return out.T


def _in_node_body(esm, geo9, w1, w2, b, out):
    out[...] = (jnp.dot(esm[...], w1[...], preferred_element_type=_f32)
                + jnp.dot(geo9[...], w2[...], preferred_element_type=_f32)
                + b[...])


def _node_proj(esm, geo9, w1, w2, b):
    blk = 2000
    grid = N_NODES // blk
    return pl.pallas_call(
        _in_node_body,
        grid=(grid,),
        in_specs=[
            pl.BlockSpec((blk, ESM_DIM), lambda i: (i, 0)),
            pl.BlockSpec((blk, 9), lambda i: (i, 0)),
            pl.BlockSpec((ESM_DIM, HID), lambda i: (0, 0)),
            pl.BlockSpec((9, HID), lambda i: (0, 0)),
            pl.BlockSpec((1, HID), lambda i: (0, 0)),
        ],
        out_specs=pl.BlockSpec((blk, HID), lambda i: (i, 0)),
        out_shape=jax.ShapeDtypeStruct((N_NODES, HID), _f32),
    )(esm, geo9, w1, w2, b)


def _edge_feat_body(e13, ev, w, b, out):
    es = jnp.dot(e13[...], w[...], preferred_element_type=_f32) + b[...]
    evv = ev[...]
    evn = jnp.sqrt(jnp.sum(evv * evv, axis=1, keepdims=True))
    out[...] = jnp.concatenate([es, evn, evv], axis=1)


def _edge_feat(e13, ev, w, b):
    blk = 2000
    grid = N_EDGES // blk
    return pl.pallas_call(
        _edge_feat_body,
        grid=(grid,),
        in_specs=[
            pl.BlockSpec((blk, 13), lambda i: (i, 0)),
            pl.BlockSpec((blk, 3), lambda i: (i, 0)),
            pl.BlockSpec((13, HID), lambda i: (0, 0)),
            pl.BlockSpec((1, HID), lambda i: (0, 0)),
        ],
        out_specs=pl.BlockSpec((blk, EFD), lambda i: (i, 0)),
        out_shape=jax.ShapeDtypeStruct((N_EDGES, EFD), _f32),
    )(e13, ev, w, b)


def _tables_body(ns, nv, s4, wcat, ts, td):
    nvv = nv[...]
    vn = jnp.sqrt(jnp.dot(nvv * nvv, s4[...], preferred_element_type=_f32))
    x = jnp.concatenate([ns[...], vn, nvv], axis=1)
    y = jnp.dot(x, wcat[...], preferred_element_type=_f32)
    z = jnp.zeros((x.shape[0], GROW - 144), _f32)
    ts[...] = jnp.concatenate([y[:, :144], z], axis=1)
    td[...] = jnp.concatenate([y[:, 144:], z], axis=1)


def _make_tables(ns, nv, s4, wcat):
    blk = 2000
    grid = N_NODES // blk
    return pl.pallas_call(
        _tables_body,
        grid=(grid,),
        in_specs=[
            pl.BlockSpec((blk, HID), lambda i: (i, 0)),
            pl.BlockSpec((blk, 12), lambda i: (i, 0)),
            pl.BlockSpec((12, 4), lambda i: (0, 0)),
            pl.BlockSpec((TROW, 2 * TROW), lambda i: (0, 0)),
        ],
        out_specs=[
            pl.BlockSpec((blk, GROW), lambda i: (i, 0)),
            pl.BlockSpec((blk, GROW), lambda i: (i, 0)),
        ],
        out_shape=[
            jax.ShapeDtypeStruct((N_NODES, GROW), _f32),
            jax.ShapeDtypeStruct((N_NODES, GROW), _f32),
        ],
    )(ns, nv, s4, wcat)


def _edge_msg_body(ga, gb, ef, wedge, lnw, lnb, bexp, out):
    t = (ga[:, :144] + gb[:, :144]
         + jnp.dot(ef[...], wedge[...], preferred_element_type=_f32))
    s = t[:, :HID]
    g = t[:, HID:HID + 4]
    v = t[:, HID + 4:TROW]
    act = _ln_gelu(s, lnw[...], lnb[...])
    gate = jnp.dot(_sigmoid(g), bexp[...], preferred_element_type=_f32)
    z = jnp.zeros((t.shape[0], 4), _f32)
    out[...] = jnp.concatenate([act, v * gate, z], axis=1)


def _edge_msgs(ga, gb, ef, wedge, lnw, lnb, bexp):
    blk = 2000
    grid = N_EDGES // blk
    return pl.pallas_call(
        _edge_msg_body,
        grid=(grid,),
        in_specs=[
            pl.BlockSpec((blk, GROW), lambda i: (i, 0)),
            pl.BlockSpec((blk, GROW), lambda i: (i, 0)),
            pl.BlockSpec((blk, EFD), lambda i: (i, 0)),
            pl.BlockSpec((EFD, TROW), lambda i: (0, 0)),
            pl.BlockSpec((1, HID), lambda i: (0, 0)),
            pl.BlockSpec((1, HID), lambda i: (0, 0)),
            pl.BlockSpec((4, 12), lambda i: (0, 0)),
        ],
        out_specs=pl.BlockSpec((blk, TROW), lambda i: (i, 0)),
        out_shape=jax.ShapeDtypeStruct((N_EDGES, TROW), _f32),
    )(ga, gb, ef, wedge, lnw, lnb, bexp)


def _update_body(ns, nv, ag0, ag1, s8, wup, wexp, lnw, lnb, bexp,
                 ns_out, nv_out):
    nsv = ns[...]
    nvv = nv[...]
    ag = ag0[...] + ag1[...]
    uv = jnp.concatenate([nvv, ag[:, HID:HID + 12]], axis=1)
    vn = jnp.sqrt(jnp.dot(uv * uv, s8[...], preferred_element_type=_f32))
    x = jnp.concatenate([nsv, ag[:, :HID], vn], axis=1)
    y = jnp.dot(x, wup[...], preferred_element_type=_f32)
    act = _ln_gelu(y[:, :HID], lnw[...], lnb[...])
    gate = jnp.dot(_sigmoid(y[:, HID:HID + 4]), bexp[...],
                   preferred_element_type=_f32)
    vout = jnp.dot(uv, wexp[...], preferred_element_type=_f32) * gate
    ns_out[...] = nsv + act
    nv_out[...] = nvv + vout


def _update(ns, nv, ag0, ag1, s8, wup, wexp, lnw, lnb, bexp):
    blk = 2000
    grid = N_NODES // blk
    return pl.pallas_call(
        _update_body,
        grid=(grid,),
        in_specs=[
            pl.BlockSpec((blk, HID), lambda i: (i, 0)),
            pl.BlockSpec((blk, 12), lambda i: (i, 0)),
            pl.BlockSpec((blk, TROW), lambda i: (i, 0)),
            pl.BlockSpec((blk, TROW), lambda i: (i, 0)),
            pl.BlockSpec((24, 8), lambda i: (0, 0)),
            pl.BlockSpec((264, EFD), lambda i: (0, 0)),
            pl.BlockSpec((24, 12), lambda i: (0, 0)),
            pl.BlockSpec((1, HID), lambda i: (0, 0)),
            pl.BlockSpec((1, HID), lambda i: (0, 0)),
            pl.BlockSpec((4, 12), lambda i: (0, 0)),
        ],
        out_specs=[
            pl.BlockSpec((blk, HID), lambda i: (i, 0)),
            pl.BlockSpec((blk, 12), lambda i: (i, 0)),
        ],
        out_shape=[
            jax.ShapeDtypeStruct((N_NODES, HID), _f32),
            jax.ShapeDtypeStruct((N_NODES, 12), _f32),
        ],
    )(ns, nv, ag0, ag1, s8, wup, wexp, lnw, lnb, bexp)


def _out_body(ns, bi, wno, bno, wgo, bgo, upd, gemb, acc, cnt):
    i = pl.program_id(0)
    nsv = ns[...]
    upd[...] = jnp.dot(nsv, wno[...], preferred_element_type=_f32) + bno[...]
    gf = jnp.dot(nsv, wgo[...], preferred_element_type=_f32) + bgo[...]
    b = bi[0]
    gid = lax.broadcasted_iota(jnp.int32, (N_GRAPHS, 1), 0)
    oh = (b == gid).astype(_f32)

    @pl.when(i == 0)
    def _():
        acc[...] = jnp.zeros_like(acc)
        cnt[...] = jnp.zeros_like(cnt)

    acc[...] += jnp.dot(oh, gf, preferred_element_type=_f32)
    cnt[...] += jnp.broadcast_to(
        jnp.sum(oh, axis=1, keepdims=True), cnt.shape)

    @pl.when(i == pl.num_programs(0) - 1)
    def _():
        gemb[...] = acc[...] / jnp.maximum(cnt[:, :1], 1.0)


def _heads(ns, bi3, wno, bno, wgo, bgo):
    blk = 2000
    grid = N_NODES // blk
    return pl.pallas_call(
        _out_body,
        grid=(grid,),
        in_specs=[
            pl.BlockSpec((blk, HID), lambda i: (i, 0)),
            pl.BlockSpec((1, 1, blk), lambda i: (i, 0, 0)),
            pl.BlockSpec((HID, ESM_DIM), lambda i: (0, 0)),
            pl.BlockSpec((1, ESM_DIM), lambda i: (0, 0)),
            pl.BlockSpec((HID, OUT_DIM), lambda i: (0, 0)),
            pl.BlockSpec((1, OUT_DIM), lambda i: (0, 0)),
        ],
        out_specs=[
            pl.BlockSpec((blk, ESM_DIM), lambda i: (i, 0)),
            pl.BlockSpec((N_GRAPHS, OUT_DIM), lambda i: (0, 0)),
        ],
        out_shape=[
            jax.ShapeDtypeStruct((N_NODES, ESM_DIM), _f32),
            jax.ShapeDtypeStruct((N_GRAPHS, OUT_DIM), _f32),
        ],
        scratch_shapes=[
            pltpu.VMEM((N_GRAPHS, OUT_DIM), _f32),
            pltpu.VMEM((N_GRAPHS, HID), _f32),
        ],
    )(ns, bi3, wno, bno, wgo, bgo)


@functools.cache
def _sc_kernels():
    mesh = plsc.VectorSubcoreMesh(core_axis_name="c", subcore_axis_name="s",
                                  num_cores=2, num_subcores=16)

    @functools.partial(
        pl.kernel,
        out_type=(jax.ShapeDtypeStruct((N_EDGES, GROW), _f32),
                  jax.ShapeDtypeStruct((N_EDGES, GROW), _f32)),
        mesh=mesh,
        scratch_types=[
            pltpu.VMEM((CH,), jnp.int32),
            pltpu.VMEM((CH,), jnp.int32),
            pltpu.VMEM((CH, GROW), _f32),
            pltpu.VMEM((CH, GROW), _f32),
            pltpu.SemaphoreType.DMA,
            pltpu.SemaphoreType.DMA,
        ],
    )
    def _sc_gather(ts_hbm, td_hbm, src_hbm, dst_hbm, oa_hbm, ob_hbm,
                   ia, ib, ra, rb, sa, sb):
        wid = lax.axis_index("s") * 2 + lax.axis_index("c")

        def body(k, carry):
            cid = k * NW + wid

            @pl.when(cid < NCHUNK)
            def _():
                base = cid * CH
                pltpu.sync_copy(src_hbm.at[pl.ds(base, CH)], ia)
                ca = pltpu.async_copy(ts_hbm.at[ia], ra, sa)
                pltpu.sync_copy(dst_hbm.at[pl.ds(base, CH)], ib)
                cb = pltpu.async_copy(td_hbm.at[ib], rb, sb)
                ca.wait()
                cb.wait()
                pltpu.sync_copy(ra, oa_hbm.at[pl.ds(base, CH)])
                pltpu.sync_copy(rb, ob_hbm.at[pl.ds(base, CH)])

            return carry

        lax.fori_loop(0, KMAX, body, 0)

    @functools.partial(
        pl.kernel,
        out_type=jax.ShapeDtypeStruct((2 * N_PAD, TROW), _f32),
        mesh=mesh,
        compiler_params=pltpu.CompilerParams(use_tc_tiling_on_sc=False),
        scratch_types=[
            pltpu.VMEM((CH,), jnp.int32),
            pltpu.VMEM((CH, TROW), _f32),
            pltpu.VMEM((ZROWS, TROW), _f32),
            pltpu.VMEM_SHARED((N_PAD, TROW), _f32),
        ],
    )
    def _sc_scatter(m_hbm, dst_hbm, out_hbm, idxv, rows, zbuf, shared):
        c = lax.axis_index("c")
        s = lax.axis_index("s")
        wid = s * 2 + c

        def zero_body(i, carry):
            for j in range(TROW // 16):
                zbuf[i, pl.ds(j * 16, 16)] = jnp.zeros((16,), _f32)
            return carry

        lax.fori_loop(0, ZROWS, zero_body, 0)

        def zcopy(i, carry):
            pltpu.sync_copy(zbuf, shared.at[pl.ds(s * RPT + i * ZROWS, ZROWS)])
            return carry

        lax.fori_loop(0, RPT // ZROWS, zcopy, 0)
        plsc.subcore_barrier()

        def body(k, carry):
            cid = k * NW + wid

            @pl.when(cid < NCHUNK)
            def _():
                base = cid * CH
                pltpu.sync_copy(dst_hbm.at[pl.ds(base, CH)], idxv)
                pltpu.sync_copy(m_hbm.at[pl.ds(base, CH)], rows)
                pltpu.sync_copy(rows, shared.at[idxv], add=True)

            return carry

        lax.fori_loop(0, KMAX, body, 0)
        plsc.subcore_barrier()
        pltpu.sync_copy(shared.at[pl.ds(s * RPT, RPT)],
                        out_hbm.at[pl.ds(c * N_PAD + s * RPT, RPT)])

    return _sc_gather, _sc_scatter


def _sc_gather_pair(ts, td, src, dst):
    return _sc_kernels()[0](ts, td, src, dst)


def _sc_scatter_add(msgs, dst):
    out = _sc_kernels()[1](msgs, dst)
    ag = out.reshape(2, N_PAD, TROW)
    return ag[0, :N_NODES], ag[1, :N_NODES]


def _msg_tables_weight(m):
    wss_t = m['Wss'].T
    wvs_t = m['Wvs'].T
    wsv_t = m['Wsv'].T
    wvv = m['Wvv']
    i3 = jnp.eye(3, dtype=_f32)

    def half(lo, vlo):
        r_s = jnp.concatenate(
            [wss_t[lo:lo + HID], wsv_t[lo:lo + HID],
             jnp.zeros((HID, 12), _f32)], axis=1)
        r_vn = jnp.concatenate(
            [wvs_t[vlo:vlo + 4], jnp.zeros((4, 16), _f32)], axis=1)
        r_v = jnp.concatenate(
            [jnp.zeros((12, EFD), _f32),
             jnp.kron(wvv[:, vlo:vlo + 4].T, i3)], axis=1)
        return jnp.concatenate([r_s, r_vn, r_v], axis=0)

    return jnp.concatenate([half(0, 0), half(HID, 4)], axis=1)


def _msg_edge_weight(m):
    i3 = jnp.eye(3, dtype=_f32)
    r_s = jnp.concatenate(
        [m['Wss'][:, 2 * HID:].T, m['Wsv'][:, 2 * HID:].T,
         jnp.zeros((HID, 12), _f32)], axis=1)
    r_n = jnp.concatenate(
        [m['Wvs'][:, 8:9].T, jnp.zeros((1, 16), _f32)], axis=1)
    r_v = jnp.concatenate(
        [jnp.zeros((3, EFD), _f32),
         jnp.kron(m['Wvv'][:, 8:9].T, i3)], axis=1)
    return jnp.concatenate([r_s, r_n, r_v], axis=0)


def _upd_weights(u):
    i3 = jnp.eye(3, dtype=_f32)
    r_s = jnp.concatenate([u['Wss'].T, u['Wsv'].T], axis=1)
    r_n = jnp.concatenate([u['Wvs'].T, jnp.zeros((8, 4), _f32)], axis=1)
    wup = jnp.concatenate([r_s, r_n], axis=0)
    wexp = jnp.kron(u['Wvv'].T, i3)
    return wup, wexp


def kernel(esm_features, geometric_features, node_coords, edge_index,
           edge_attr, batch_index, params):
    f32 = _f32
    i3 = jnp.eye(3, dtype=f32)
    s4 = jnp.kron(jnp.eye(4, dtype=f32), jnp.ones((3, 1), f32))
    s8 = jnp.kron(jnp.eye(8, dtype=f32), jnp.ones((3, 1), f32))
    bexp = jnp.kron(jnp.eye(4, dtype=f32), jnp.ones((1, 3), f32))

    cb = geometric_features[:, 3:6]
    geo9 = geometric_features[:, 6:15]
    node_v = _node_vectors(node_coords, cb)

    node_s = _node_proj(
        esm_features, geo9,
        params['node_proj_W'][:, :ESM_DIM].T,
        params['node_proj_W'][:, ESM_DIM:].T,
        params['node_proj_b'][None, :])

    e13 = jnp.concatenate([edge_attr[:, 0:2], edge_attr[:, 5:]], axis=1)
    ev = edge_attr[:, 2:5]
    ef = _edge_feat(e13, ev, params['edge_proj_W'].T,
                    params['edge_proj_b'][None, :])

    src = edge_index[0]
    dst = edge_index[1]

    for lp in params['layers']:
        m = lp['msg']
        u = lp['upd']
        wcat = _msg_tables_weight(m)
        wedge = _msg_edge_weight(m)
        wup, wexp = _upd_weights(u)

        ts, td = _make_tables(node_s, node_v, s4, wcat)
        ga, gb = _sc_gather_pair(ts, td, src, dst)
        msgs = _edge_msgs(ga, gb, ef, wedge,
                          m['ln_w'][None, :], m['ln_b'][None, :], bexp)
        ag0, ag1 = _sc_scatter_add(msgs, dst)
        node_s, node_v = _update(node_s, node_v, ag0, ag1, s8, wup, wexp,
                                 u['ln_w'][None, :], u['ln_b'][None, :], bexp)

    bi3 = batch_index.reshape(5, 1, 2000)
    upd, gemb = _heads(node_s, bi3,
                       params['node_out_W'].T, params['node_out_b'][None, :],
                       params['graph_out_W'].T, params['graph_out_b'][None, :])
    return gemb, upd
